# baseline (speedup 1.0000x reference)
import functools

import jax
import jax.numpy as jnp
from jax import lax
from jax.experimental import pallas as pl
from jax.experimental.pallas import tpu as pltpu
from jax.experimental.pallas import tpu_sc as plsc

N = 10000
D = 128
E = 320000
BN_EPS = 1e-5

NC = 2
NS = 16
NW = NC * NS
EPW = E // NW
CH = 80
NCH = EPW // CH
DEGP = 10240
SEG = DEGP // NS
NP = 10240
ROWS_PT = NP // NS
ZR = 32
NB = 4
GLA = 2
RB = 1000

_LANES = 16


def _sc_mesh():
    return plsc.VectorSubcoreMesh(
        core_axis_name="c", subcore_axis_name="s",
        num_cores=NC, num_subcores=NS)


@functools.partial(
    pl.kernel,
    out_type=jax.ShapeDtypeStruct((NW, 1, SEG), jnp.float32),
    mesh=_sc_mesh(),
    scratch_types=[
        pltpu.VMEM((1, EPW), jnp.int32),
        pltpu.VMEM((1, DEGP), jnp.float32),
        pltpu.VMEM((NS, 1, SEG), jnp.float32),
        pltpu.VMEM((1, SEG), jnp.float32),
        pltpu.VMEM_SHARED((NS, 1, DEGP), jnp.float32),
    ],
    compiler_params=pltpu.CompilerParams(needs_layout_passes=False),
)
def _deg_call(dst_hbm, out_hbm, idx_v, hist_v, red_v, acc_v, part_sh):
    c = lax.axis_index("c")
    s = lax.axis_index("s")
    wid = c * NS + s

    zero16 = jnp.zeros((_LANES,), jnp.float32)
    zero16i = jnp.zeros((_LANES,), jnp.int32)
    ones16 = jnp.ones((_LANES,), jnp.float32)

    def zbody(i, carry):
        hist_v[0, pl.ds(i * _LANES, _LANES)] = zero16
        return carry
    lax.fori_loop(0, DEGP // _LANES, zbody, 0)

    pltpu.sync_copy(dst_hbm.at[wid], idx_v)

    def sbody(j, carry):
        idx = idx_v[0, pl.ds(j * _LANES, _LANES)]
        plsc.addupdate_scatter(hist_v, [zero16i, idx], ones16)
        return carry
    lax.fori_loop(0, EPW // _LANES, sbody, 0)

    pltpu.sync_copy(hist_v, part_sh.at[s])
    plsc.subcore_barrier()

    for t in range(NS):
        pltpu.sync_copy(part_sh.at[t, :, pl.ds(s * SEG, SEG)], red_v.at[t])

    def rbody(i, carry):
        v = red_v[0, 0, pl.ds(i * _LANES, _LANES)]
        for t in range(1, NS):
            v = v + red_v[t, 0, pl.ds(i * _LANES, _LANES)]
        acc_v[0, pl.ds(i * _LANES, _LANES)] = v
        return carry
    lax.fori_loop(0, SEG // _LANES, rbody, 0)

    pltpu.sync_copy(acc_v, out_hbm.at[wid])


def _tca_body(x_ref, w_ref, degp_ref, h2_ref):
    deg = degp_ref[:, 0] + degp_ref[:, 1] + 1.0
    dis = lax.rsqrt(deg)
    h = jnp.dot(x_ref[...], w_ref[...],
                preferred_element_type=jnp.float32,
                precision=lax.Precision.HIGHEST)
    h2_ref[...] = h * dis[:, None]


def _tca_call(x, W, degp):
    return pl.pallas_call(
        _tca_body,
        grid=(N // RB,),
        in_specs=[
            pl.BlockSpec((RB, D), lambda j: (j, 0)),
            pl.BlockSpec((D, D), lambda j: (0, 0)),
            pl.BlockSpec((RB, NC), lambda j: (j, 0)),
        ],
        out_specs=pl.BlockSpec((RB, D), lambda j: (j, 0)),
        out_shape=jax.ShapeDtypeStruct((N, D), jnp.float32),
    )(x, W, degp)


@functools.partial(
    pl.kernel,
    out_type=jax.ShapeDtypeStruct((NC, NP, D), jnp.float32),
    mesh=_sc_mesh(),
    scratch_types=(
        [pltpu.VMEM((1, CH), jnp.int32) for _ in range(NB)]
        + [pltpu.VMEM((1, CH), jnp.int32) for _ in range(NB)]
        + [pltpu.VMEM((CH, D), jnp.float32) for _ in range(NB)]
        + [
            pltpu.VMEM((ZR, D), jnp.float32),
            pltpu.VMEM_SHARED((NP, D), jnp.float32),
        ]
        + [pltpu.SemaphoreType.DMA for _ in range(4 * NB)]
    ),
    compiler_params=pltpu.CompilerParams(needs_layout_passes=False),
)
def _msg_call(h2_hbm, src_hbm, dst_hbm, out_hbm, *scr):
    si = list(scr[0:NB])
    di = list(scr[NB:2 * NB])
    rw = list(scr[2 * NB:3 * NB])
    zrow = scr[3 * NB]
    acc_sh = scr[3 * NB + 1]
    sg = list(scr[3 * NB + 2:3 * NB + 2 + NB])
    ss = list(scr[3 * NB + 2 + NB:3 * NB + 2 + 2 * NB])
    sd = list(scr[3 * NB + 2 + 2 * NB:3 * NB + 2 + 3 * NB])
    sc = list(scr[3 * NB + 2 + 3 * NB:3 * NB + 2 + 4 * NB])

    c = lax.axis_index("c")
    s = lax.axis_index("s")
    wid = c * NS + s
    base = wid * NCH

    zero16 = jnp.zeros((_LANES,), jnp.float32)

    def zbody(i, carry):
        for jj in range(D // _LANES):
            zrow[i, pl.ds(jj * _LANES, _LANES)] = zero16
        return carry
    lax.fori_loop(0, ZR, zbody, 0)
    for m in range(ROWS_PT // ZR):
        pltpu.sync_copy(zrow, acc_sh.at[pl.ds(s * ROWS_PT + m * ZR, ZR)])
    plsc.subcore_barrier()

    def _gather(k, z):
        pltpu.make_async_copy(src_hbm.at[base + k], si[z], ss[z]).wait()
        pltpu.async_copy(h2_hbm.at[si[z].at[0]], rw[z], sg[z])

    for k in range(NB):
        pltpu.async_copy(src_hbm.at[base + k], si[k], ss[k])
    for k in range(GLA):
        pltpu.async_copy(dst_hbm.at[base + k], di[k], sd[k])
    for k in range(GLA):
        _gather(k, k)

    def _slot(j, x):
        z = (x + GLA) % NB
        pltpu.make_async_copy(h2_hbm.at[si[x].at[0]], rw[x], sg[x]).wait()

        @pl.when(j + NB < NCH)
        def _():
            pltpu.async_copy(src_hbm.at[base + j + NB], si[x], ss[x])
        pltpu.make_async_copy(dst_hbm.at[base + j], di[x], sd[x]).wait()
        pltpu.async_copy(rw[x], acc_sh.at[di[x].at[0]], sc[x], add=True)

        @pl.when(j >= 2)
        def _():
            pltpu.make_async_copy(rw[z], acc_sh.at[di[z].at[0]], sc[z]).wait()

        @pl.when(j + GLA < NCH)
        def _():
            pltpu.async_copy(dst_hbm.at[base + j + GLA], di[z], sd[z])
            _gather(j + GLA, z)

    def rot(k, carry):
        for x in range(NB):
            _slot(NB * k + x, x)
        return carry
    lax.fori_loop(0, NCH // NB, rot, 0)
    for j in range(NB * (NCH // NB), NCH):
        _slot(j, j % NB)

    for j in (NCH - 2, NCH - 1):
        x = j % NB
        pltpu.make_async_copy(rw[x], acc_sh.at[di[x].at[0]], sc[x]).wait()

    plsc.subcore_barrier()
    pltpu.sync_copy(acc_sh.at[pl.ds(s * ROWS_PT, ROWS_PT)],
                    out_hbm.at[c, pl.ds(s * ROWS_PT, ROWS_PT)])


def _tcb_body(msg_ref, h2_ref, degp_ref, b_ref, gamma_ref, beta_ref,
              out_ref, pre_ref, sum_ref, sumsq_ref):
    p = pl.program_id(0)
    j = pl.program_id(1)

    @pl.when(p == 0)
    def _():
        deg = degp_ref[:, 0] + degp_ref[:, 1] + 1.0
        dis = lax.rsqrt(deg)
        pre = ((msg_ref[0] + msg_ref[1] + h2_ref[...]) * dis[:, None]
               + b_ref[...])
        pre_ref[pl.ds(j * RB, RB), :] = pre
        ps = jnp.sum(pre, axis=0, keepdims=True)
        pss = jnp.sum(pre * pre, axis=0, keepdims=True)

        @pl.when(j == 0)
        def _():
            sum_ref[...] = ps
            sumsq_ref[...] = pss

        @pl.when(j != 0)
        def _():
            sum_ref[...] = sum_ref[...] + ps
            sumsq_ref[...] = sumsq_ref[...] + pss

    @pl.when(p == 1)
    def _():
        mean = sum_ref[...] * (1.0 / N)
        var = sumsq_ref[...] * (1.0 / N) - mean * mean
        inv = lax.rsqrt(var + BN_EPS)
        pre = pre_ref[pl.ds(j * RB, RB), :]
        y = (pre - mean) * (inv * gamma_ref[...]) + beta_ref[...]
        out_ref[...] = jnp.maximum(y, 0.0)


def _tcb_call(msg, h2, degp, b2, gamma2, beta2):
    return pl.pallas_call(
        _tcb_body,
        grid=(2, N // RB),
        in_specs=[
            pl.BlockSpec((NC, RB, D), lambda p, j: (0, j * (1 - p), 0)),
            pl.BlockSpec((RB, D), lambda p, j: (j * (1 - p), 0)),
            pl.BlockSpec((RB, NC), lambda p, j: (j * (1 - p), 0)),
            pl.BlockSpec((1, D), lambda p, j: (0, 0)),
            pl.BlockSpec((1, D), lambda p, j: (0, 0)),
            pl.BlockSpec((1, D), lambda p, j: (0, 0)),
        ],
        out_specs=pl.BlockSpec((RB, D), lambda p, j: (j, 0)),
        out_shape=jax.ShapeDtypeStruct((N, D), jnp.float32),
        scratch_shapes=[
            pltpu.VMEM((N, D), jnp.float32),
            pltpu.VMEM((1, D), jnp.float32),
            pltpu.VMEM((1, D), jnp.float32),
        ],
    )(msg, h2, degp, b2, gamma2, beta2)


def kernel(x, edge_index, W, b, gamma, beta):
    src = edge_index[0].astype(jnp.int32)
    dst = edge_index[1].astype(jnp.int32)

    degs = _deg_call(dst.reshape(NW, 1, EPW))
    degp = degs.reshape(NC, DEGP)[:, :N].T
    h2 = _tca_call(x, W, degp)
    msg = _msg_call(h2,
                    src.reshape(E // CH, 1, CH),
                    dst.reshape(E // CH, 1, CH))
    return _tcb_call(msg, h2, degp, b.reshape(1, D),
                     gamma.reshape(1, D), beta.reshape(1, D))


# speedup vs baseline: 1.0717x; 1.0717x over previous
import functools

import jax
import jax.numpy as jnp
from jax import lax
from jax.experimental import pallas as pl
from jax.experimental.pallas import tpu as pltpu
from jax.experimental.pallas import tpu_sc as plsc

N = 10000
D = 128
E = 320000
BN_EPS = 1e-5

NC = 2
NS = 16
NW = NC * NS
EPW = E // NW
CH = 80
NCH = EPW // CH
DEGP = 10240
SEG = DEGP // NS
NP = 10240
ROWS_PT = NP // NS
ZR = 32
NB = 4
GLA = 3
RB = 1000

_LANES = 16


def _sc_mesh():
    return plsc.VectorSubcoreMesh(
        core_axis_name="c", subcore_axis_name="s",
        num_cores=NC, num_subcores=NS)


@functools.partial(
    pl.kernel,
    out_type=jax.ShapeDtypeStruct((NW, 1, SEG), jnp.float32),
    mesh=_sc_mesh(),
    scratch_types=[
        pltpu.VMEM((1, EPW), jnp.int32),
        pltpu.VMEM((1, DEGP), jnp.float32),
        pltpu.VMEM((NS, 1, SEG), jnp.float32),
        pltpu.VMEM((1, SEG), jnp.float32),
        pltpu.VMEM_SHARED((NS, 1, DEGP), jnp.float32),
    ],
    compiler_params=pltpu.CompilerParams(needs_layout_passes=False),
)
def _deg_call(dst_hbm, out_hbm, idx_v, hist_v, red_v, acc_v, part_sh):
    c = lax.axis_index("c")
    s = lax.axis_index("s")
    wid = c * NS + s

    zero16 = jnp.zeros((_LANES,), jnp.float32)
    zero16i = jnp.zeros((_LANES,), jnp.int32)
    ones16 = jnp.ones((_LANES,), jnp.float32)

    def zbody(i, carry):
        hist_v[0, pl.ds(i * _LANES, _LANES)] = zero16
        return carry
    lax.fori_loop(0, DEGP // _LANES, zbody, 0)

    pltpu.sync_copy(dst_hbm.at[wid], idx_v)

    def sbody(j, carry):
        idx = idx_v[0, pl.ds(j * _LANES, _LANES)]
        plsc.addupdate_scatter(hist_v, [zero16i, idx], ones16)
        return carry
    lax.fori_loop(0, EPW // _LANES, sbody, 0)

    pltpu.sync_copy(hist_v, part_sh.at[s])
    plsc.subcore_barrier()

    for t in range(NS):
        pltpu.sync_copy(part_sh.at[t, :, pl.ds(s * SEG, SEG)], red_v.at[t])

    def rbody(i, carry):
        v = red_v[0, 0, pl.ds(i * _LANES, _LANES)]
        for t in range(1, NS):
            v = v + red_v[t, 0, pl.ds(i * _LANES, _LANES)]
        acc_v[0, pl.ds(i * _LANES, _LANES)] = v
        return carry
    lax.fori_loop(0, SEG // _LANES, rbody, 0)

    pltpu.sync_copy(acc_v, out_hbm.at[wid])


def _tca_body(x_ref, w_ref, degp_ref, h2_ref):
    deg = degp_ref[:, 0] + degp_ref[:, 1] + 1.0
    dis = lax.rsqrt(deg)
    h = jnp.dot(x_ref[...], w_ref[...],
                preferred_element_type=jnp.float32,
                precision=lax.Precision.HIGHEST)
    h2_ref[...] = h * dis[:, None]


def _tca_call(x, W, degp):
    return pl.pallas_call(
        _tca_body,
        grid=(N // RB,),
        in_specs=[
            pl.BlockSpec((RB, D), lambda j: (j, 0)),
            pl.BlockSpec((D, D), lambda j: (0, 0)),
            pl.BlockSpec((RB, NC), lambda j: (j, 0)),
        ],
        out_specs=pl.BlockSpec((RB, D), lambda j: (j, 0)),
        out_shape=jax.ShapeDtypeStruct((N, D), jnp.float32),
    )(x, W, degp)


@functools.partial(
    pl.kernel,
    out_type=jax.ShapeDtypeStruct((NC, NP, D), jnp.float32),
    mesh=_sc_mesh(),
    scratch_types=(
        [pltpu.VMEM((1, CH), jnp.int32) for _ in range(NB)]
        + [pltpu.VMEM((1, CH), jnp.int32) for _ in range(NB)]
        + [pltpu.VMEM((CH, D), jnp.float32) for _ in range(NB)]
        + [
            pltpu.VMEM((ZR, D), jnp.float32),
            pltpu.VMEM_SHARED((NP, D), jnp.float32),
        ]
        + [pltpu.SemaphoreType.DMA for _ in range(4 * NB)]
    ),
    compiler_params=pltpu.CompilerParams(needs_layout_passes=False),
)
def _msg_call(h2_hbm, src_hbm, dst_hbm, out_hbm, *scr):
    si = list(scr[0:NB])
    di = list(scr[NB:2 * NB])
    rw = list(scr[2 * NB:3 * NB])
    zrow = scr[3 * NB]
    acc_sh = scr[3 * NB + 1]
    sg = list(scr[3 * NB + 2:3 * NB + 2 + NB])
    ss = list(scr[3 * NB + 2 + NB:3 * NB + 2 + 2 * NB])
    sd = list(scr[3 * NB + 2 + 2 * NB:3 * NB + 2 + 3 * NB])
    sc = list(scr[3 * NB + 2 + 3 * NB:3 * NB + 2 + 4 * NB])

    c = lax.axis_index("c")
    s = lax.axis_index("s")
    wid = c * NS + s
    base = wid * NCH

    zero16 = jnp.zeros((_LANES,), jnp.float32)

    def zbody(i, carry):
        for jj in range(D // _LANES):
            zrow[i, pl.ds(jj * _LANES, _LANES)] = zero16
        return carry
    lax.fori_loop(0, ZR, zbody, 0)
    for m in range(ROWS_PT // ZR):
        pltpu.sync_copy(zrow, acc_sh.at[pl.ds(s * ROWS_PT + m * ZR, ZR)])
    plsc.subcore_barrier()

    def _gather(k, z):
        pltpu.make_async_copy(src_hbm.at[base + k], si[z], ss[z]).wait()
        pltpu.async_copy(h2_hbm.at[si[z].at[0]], rw[z], sg[z])

    for k in range(NB):
        pltpu.async_copy(src_hbm.at[base + k], si[k], ss[k])
    for k in range(GLA):
        pltpu.async_copy(dst_hbm.at[base + k], di[k], sd[k])
    for k in range(GLA):
        _gather(k, k)

    def _slot(j, x):
        z = (x + GLA) % NB
        pltpu.make_async_copy(h2_hbm.at[si[x].at[0]], rw[x], sg[x]).wait()

        @pl.when(j + NB < NCH)
        def _():
            pltpu.async_copy(src_hbm.at[base + j + NB], si[x], ss[x])
        pltpu.make_async_copy(dst_hbm.at[base + j], di[x], sd[x]).wait()
        pltpu.async_copy(rw[x], acc_sh.at[di[x].at[0]], sc[x], add=True)

        @pl.when(j >= NB - GLA)
        def _():
            pltpu.make_async_copy(rw[z], acc_sh.at[di[z].at[0]], sc[z]).wait()

        @pl.when(j + GLA < NCH)
        def _():
            pltpu.async_copy(dst_hbm.at[base + j + GLA], di[z], sd[z])
            _gather(j + GLA, z)

    def rot(k, carry):
        for x in range(NB):
            _slot(NB * k + x, x)
        return carry
    lax.fori_loop(0, NCH // NB, rot, 0)
    for j in range(NB * (NCH // NB), NCH):
        _slot(j, j % NB)

    for j in range(NCH - (NB - GLA), NCH):
        x = j % NB
        pltpu.make_async_copy(rw[x], acc_sh.at[di[x].at[0]], sc[x]).wait()

    plsc.subcore_barrier()
    pltpu.sync_copy(acc_sh.at[pl.ds(s * ROWS_PT, ROWS_PT)],
                    out_hbm.at[c, pl.ds(s * ROWS_PT, ROWS_PT)])


def _tcb_body(msg_ref, h2_ref, degp_ref, b_ref, gamma_ref, beta_ref,
              out_ref, pre_ref, sum_ref, sumsq_ref):
    p = pl.program_id(0)
    j = pl.program_id(1)

    @pl.when(p == 0)
    def _():
        deg = degp_ref[:, 0] + degp_ref[:, 1] + 1.0
        dis = lax.rsqrt(deg)
        pre = ((msg_ref[0] + msg_ref[1] + h2_ref[...]) * dis[:, None]
               + b_ref[...])
        pre_ref[pl.ds(j * RB, RB), :] = pre
        ps = jnp.sum(pre, axis=0, keepdims=True)
        pss = jnp.sum(pre * pre, axis=0, keepdims=True)

        @pl.when(j == 0)
        def _():
            sum_ref[...] = ps
            sumsq_ref[...] = pss

        @pl.when(j != 0)
        def _():
            sum_ref[...] = sum_ref[...] + ps
            sumsq_ref[...] = sumsq_ref[...] + pss

    @pl.when(p == 1)
    def _():
        mean = sum_ref[...] * (1.0 / N)
        var = sumsq_ref[...] * (1.0 / N) - mean * mean
        inv = lax.rsqrt(var + BN_EPS)
        pre = pre_ref[pl.ds(j * RB, RB), :]
        y = (pre - mean) * (inv * gamma_ref[...]) + beta_ref[...]
        out_ref[...] = jnp.maximum(y, 0.0)


def _tcb_call(msg, h2, degp, b2, gamma2, beta2):
    return pl.pallas_call(
        _tcb_body,
        grid=(2, N // RB),
        in_specs=[
            pl.BlockSpec((NC, RB, D), lambda p, j: (0, j * (1 - p), 0)),
            pl.BlockSpec((RB, D), lambda p, j: (j * (1 - p), 0)),
            pl.BlockSpec((RB, NC), lambda p, j: (j * (1 - p), 0)),
            pl.BlockSpec((1, D), lambda p, j: (0, 0)),
            pl.BlockSpec((1, D), lambda p, j: (0, 0)),
            pl.BlockSpec((1, D), lambda p, j: (0, 0)),
        ],
        out_specs=pl.BlockSpec((RB, D), lambda p, j: (j, 0)),
        out_shape=jax.ShapeDtypeStruct((N, D), jnp.float32),
        scratch_shapes=[
            pltpu.VMEM((N, D), jnp.float32),
            pltpu.VMEM((1, D), jnp.float32),
            pltpu.VMEM((1, D), jnp.float32),
        ],
    )(msg, h2, degp, b2, gamma2, beta2)


def kernel(x, edge_index, W, b, gamma, beta):
    src = edge_index[0].astype(jnp.int32)
    dst = edge_index[1].astype(jnp.int32)

    degs = _deg_call(dst.reshape(NW, 1, EPW))
    degp = degs.reshape(NC, DEGP)[:, :N].T
    h2 = _tca_call(x, W, degp)
    msg = _msg_call(h2,
                    src.reshape(E // CH, 1, CH),
                    dst.reshape(E // CH, 1, CH))
    return _tcb_call(msg, h2, degp, b.reshape(1, D),
                     gamma.reshape(1, D), beta.reshape(1, D))


# speedup vs baseline: 1.1124x; 1.0379x over previous
import functools

import jax
import jax.numpy as jnp
from jax import lax
from jax.experimental import pallas as pl
from jax.experimental.pallas import tpu as pltpu
from jax.experimental.pallas import tpu_sc as plsc

N = 10000
D = 128
E = 320000
BN_EPS = 1e-5

NC = 2
NS = 16
NW = NC * NS
EPW = E // NW
CH = 80
NCH = EPW // CH
DEGP = 10240
SEG = DEGP // NS
NP = 10240
ROWS_PT = NP // NS
ZR = 32
NB = 4
RB = 1000

_LANES = 16


def _sc_mesh():
    return plsc.VectorSubcoreMesh(
        core_axis_name="c", subcore_axis_name="s",
        num_cores=NC, num_subcores=NS)


@functools.partial(
    pl.kernel,
    out_type=jax.ShapeDtypeStruct((NW, 1, SEG), jnp.float32),
    mesh=_sc_mesh(),
    scratch_types=[
        pltpu.VMEM((1, EPW), jnp.int32),
        pltpu.VMEM((1, DEGP), jnp.float32),
        pltpu.VMEM((NS, 1, SEG), jnp.float32),
        pltpu.VMEM((1, SEG), jnp.float32),
        pltpu.VMEM_SHARED((NS, 1, DEGP), jnp.float32),
    ],
    compiler_params=pltpu.CompilerParams(needs_layout_passes=False),
)
def _deg_call(dst_hbm, out_hbm, idx_v, hist_v, red_v, acc_v, part_sh):
    c = lax.axis_index("c")
    s = lax.axis_index("s")
    wid = c * NS + s

    zero16 = jnp.zeros((_LANES,), jnp.float32)
    zero16i = jnp.zeros((_LANES,), jnp.int32)
    ones16 = jnp.ones((_LANES,), jnp.float32)

    def zbody(i, carry):
        hist_v[0, pl.ds(i * _LANES, _LANES)] = zero16
        return carry
    lax.fori_loop(0, DEGP // _LANES, zbody, 0)

    pltpu.sync_copy(dst_hbm.at[wid], idx_v)

    def sbody(j, carry):
        idx = idx_v[0, pl.ds(j * _LANES, _LANES)]
        plsc.addupdate_scatter(hist_v, [zero16i, idx], ones16)
        return carry
    lax.fori_loop(0, EPW // _LANES, sbody, 0)

    pltpu.sync_copy(hist_v, part_sh.at[s])
    plsc.subcore_barrier()

    for t in range(NS):
        pltpu.sync_copy(part_sh.at[t, :, pl.ds(s * SEG, SEG)], red_v.at[t])

    def rbody(i, carry):
        v = red_v[0, 0, pl.ds(i * _LANES, _LANES)]
        for t in range(1, NS):
            v = v + red_v[t, 0, pl.ds(i * _LANES, _LANES)]
        acc_v[0, pl.ds(i * _LANES, _LANES)] = v
        return carry
    lax.fori_loop(0, SEG // _LANES, rbody, 0)

    pltpu.sync_copy(acc_v, out_hbm.at[wid])


def _tca_body(x_ref, w_ref, degp_ref, h2_ref):
    deg = degp_ref[:, 0] + degp_ref[:, 1] + 1.0
    dis = lax.rsqrt(deg)
    h = jnp.dot(x_ref[...], w_ref[...],
                preferred_element_type=jnp.float32,
                precision=lax.Precision.HIGHEST)
    h2_ref[...] = h * dis[:, None]


def _tca_call(x, W, degp):
    return pl.pallas_call(
        _tca_body,
        grid=(N // RB,),
        in_specs=[
            pl.BlockSpec((RB, D), lambda j: (j, 0)),
            pl.BlockSpec((D, D), lambda j: (0, 0)),
            pl.BlockSpec((RB, NC), lambda j: (j, 0)),
        ],
        out_specs=pl.BlockSpec((RB, D), lambda j: (j, 0)),
        out_shape=jax.ShapeDtypeStruct((N, D), jnp.float32),
    )(x, W, degp)


@functools.partial(
    pl.kernel,
    out_type=jax.ShapeDtypeStruct((NC, NP, D), jnp.float32),
    mesh=_sc_mesh(),
    scratch_types=(
        [pltpu.VMEM((1, CH), jnp.int32) for _ in range(NB)]
        + [pltpu.VMEM((1, CH), jnp.int32) for _ in range(NB)]
        + [pltpu.VMEM((CH, D), jnp.float32) for _ in range(NB)]
        + [
            pltpu.VMEM((ZR, D), jnp.float32),
            pltpu.VMEM_SHARED((NP, D), jnp.float32),
        ]
        + [pltpu.SemaphoreType.DMA for _ in range(3 * NB)]
    ),
    compiler_params=pltpu.CompilerParams(needs_layout_passes=False),
)
def _msg_call(h2_hbm, src_hbm, dst_hbm, out_hbm,
              si0, si1, si2, si3, di0, di1, di2, di3,
              rw0, rw1, rw2, rw3, zrow, acc_sh,
              sg0, sg1, sg2, sg3, ss0, ss1, ss2, ss3,
              sd0, sd1, sd2, sd3):
    c = lax.axis_index("c")
    s = lax.axis_index("s")
    wid = c * NS + s
    base = wid * NCH

    si = [si0, si1, si2, si3]
    di = [di0, di1, di2, di3]
    rw = [rw0, rw1, rw2, rw3]
    sg = [sg0, sg1, sg2, sg3]
    ss = [ss0, ss1, ss2, ss3]
    sd = [sd0, sd1, sd2, sd3]

    zero16 = jnp.zeros((_LANES,), jnp.float32)

    def zbody(i, carry):
        for jj in range(D // _LANES):
            zrow[i, pl.ds(jj * _LANES, _LANES)] = zero16
        return carry
    lax.fori_loop(0, ZR, zbody, 0)
    for m in range(ROWS_PT // ZR):
        pltpu.sync_copy(zrow, acc_sh.at[pl.ds(s * ROWS_PT + m * ZR, ZR)])
    plsc.subcore_barrier()

    for x in range(NB):
        pltpu.async_copy(src_hbm.at[base + x], si[x], ss[x])
        pltpu.async_copy(dst_hbm.at[base + x], di[x], sd[x])
    for x in range(NB):
        pltpu.make_async_copy(src_hbm.at[base + x], si[x], ss[x]).wait()
        pltpu.async_copy(h2_hbm.at[si[x].at[0]], rw[x], sg[x])

    def _slot(j, x):
        pltpu.make_async_copy(h2_hbm.at[si[x].at[0]], rw[x], sg[x]).wait()

        @pl.when(j + NB < NCH)
        def _():
            pltpu.async_copy(src_hbm.at[base + j + NB], si[x], ss[x])
        pltpu.make_async_copy(dst_hbm.at[base + j], di[x], sd[x]).wait()
        pltpu.sync_copy(rw[x], acc_sh.at[di[x].at[0]], add=True)

        @pl.when(j + NB < NCH)
        def _():
            pltpu.async_copy(dst_hbm.at[base + j + NB], di[x], sd[x])
            pltpu.make_async_copy(src_hbm.at[base + j + NB], si[x], ss[x]).wait()
            pltpu.async_copy(h2_hbm.at[si[x].at[0]], rw[x], sg[x])

    def rot(k, carry):
        for x in range(NB):
            _slot(NB * k + x, x)
        return carry
    lax.fori_loop(0, NCH // NB, rot, 0)
    for j in range(NB * (NCH // NB), NCH):
        _slot(j, j % NB)

    plsc.subcore_barrier()
    pltpu.sync_copy(acc_sh.at[pl.ds(s * ROWS_PT, ROWS_PT)],
                    out_hbm.at[c, pl.ds(s * ROWS_PT, ROWS_PT)])


def _tcb_body(msg_ref, h2_ref, degp_ref, b_ref, gamma_ref, beta_ref,
              out_ref, pre_ref, sum_ref, sumsq_ref):
    p = pl.program_id(0)
    j = pl.program_id(1)

    @pl.when(p == 0)
    def _():
        deg = degp_ref[:, 0] + degp_ref[:, 1] + 1.0
        dis = lax.rsqrt(deg)
        pre = ((msg_ref[0] + msg_ref[1] + h2_ref[...]) * dis[:, None]
               + b_ref[...])
        pre_ref[pl.ds(j * RB, RB), :] = pre
        ps = jnp.sum(pre, axis=0, keepdims=True)
        pss = jnp.sum(pre * pre, axis=0, keepdims=True)

        @pl.when(j == 0)
        def _():
            sum_ref[...] = ps
            sumsq_ref[...] = pss

        @pl.when(j != 0)
        def _():
            sum_ref[...] = sum_ref[...] + ps
            sumsq_ref[...] = sumsq_ref[...] + pss

    @pl.when(p == 1)
    def _():
        mean = sum_ref[...] * (1.0 / N)
        var = sumsq_ref[...] * (1.0 / N) - mean * mean
        inv = lax.rsqrt(var + BN_EPS)
        pre = pre_ref[pl.ds(j * RB, RB), :]
        y = (pre - mean) * (inv * gamma_ref[...]) + beta_ref[...]
        out_ref[...] = jnp.maximum(y, 0.0)


def _tcb_call(msg, h2, degp, b2, gamma2, beta2):
    return pl.pallas_call(
        _tcb_body,
        grid=(2, N // RB),
        in_specs=[
            pl.BlockSpec((NC, RB, D), lambda p, j: (0, j * (1 - p), 0)),
            pl.BlockSpec((RB, D), lambda p, j: (j * (1 - p), 0)),
            pl.BlockSpec((RB, NC), lambda p, j: (j * (1 - p), 0)),
            pl.BlockSpec((1, D), lambda p, j: (0, 0)),
            pl.BlockSpec((1, D), lambda p, j: (0, 0)),
            pl.BlockSpec((1, D), lambda p, j: (0, 0)),
        ],
        out_specs=pl.BlockSpec((RB, D), lambda p, j: (j, 0)),
        out_shape=jax.ShapeDtypeStruct((N, D), jnp.float32),
        scratch_shapes=[
            pltpu.VMEM((N, D), jnp.float32),
            pltpu.VMEM((1, D), jnp.float32),
            pltpu.VMEM((1, D), jnp.float32),
        ],
    )(msg, h2, degp, b2, gamma2, beta2)


def kernel(x, edge_index, W, b, gamma, beta):
    src = edge_index[0].astype(jnp.int32)
    dst = edge_index[1].astype(jnp.int32)

    degs = _deg_call(dst.reshape(NW, 1, EPW))
    degp = degs.reshape(NC, DEGP)[:, :N].T
    h2 = _tca_call(x, W, degp)
    msg = _msg_call(h2,
                    src.reshape(E // CH, 1, CH),
                    dst.reshape(E // CH, 1, CH))
    return _tcb_call(msg, h2, degp, b.reshape(1, D),
                     gamma.reshape(1, D), beta.reshape(1, D))
